# baseline (speedup 1.0000x reference)
import functools

import jax
import jax.numpy as jnp
from jax import lax
from jax.experimental import pallas as pl
from jax.experimental.pallas import tpu as pltpu
from jax.experimental.pallas import tpu_sc as plsc

_CHUNK = 128
_LANES = 16
_ABLK = 5000


@functools.cache
def _build_widen(V: int, D: int):
    def body(x_ref, o_ref):
        x = x_ref[...]
        o_ref[...] = jnp.concatenate([x, x], axis=1)

    return pl.pallas_call(
        body,
        grid=(V // _ABLK,),
        in_specs=[pl.BlockSpec((_ABLK, D), lambda i: (i, 0))],
        out_specs=pl.BlockSpec((_ABLK, 2 * D), lambda i: (i, 0)),
        out_shape=jax.ShapeDtypeStruct((V, 2 * D), jnp.float32),
    )


@functools.cache
def _build_sc_gather(B: int, V: int, D: int):
    info = plsc.get_sparse_core_info()
    nc, ns = info.num_cores, info.num_subcores
    nw = nc * ns
    assert B % (8 * nw) == 0, "batch must split 8-aligned across subcores"
    b_per_w = B // nw
    assert b_per_w % _CHUNK == 0

    mesh = plsc.VectorSubcoreMesh(core_axis_name="c", subcore_axis_name="s")

    @functools.partial(
        pl.kernel,
        mesh=mesh,
        out_type=jax.ShapeDtypeStruct((B, D), jnp.float32),
        scratch_types=[
            pltpu.VMEM((b_per_w,), jnp.int32),
            pltpu.VMEM((_CHUNK, 2 * D), jnp.float32),
            pltpu.VMEM((b_per_w, D), jnp.float32),
            pltpu.SemaphoreType.DMA,
        ],
    )
    def sc_gather(n_id_hbm, wide_hbm, out_hbm, idx_v, pad_v, rows_v, sem):
        wid = lax.axis_index("s") * nc + lax.axis_index("c")
        base = wid * b_per_w
        pltpu.sync_copy(n_id_hbm.at[pl.ds(base, b_per_w)], idx_v)

        for k in range(b_per_w // _CHUNK):
            pltpu.async_copy(
                wide_hbm.at[idx_v.at[pl.ds(k * _CHUNK, _CHUNK)]], pad_v, sem)
            pltpu.make_async_copy(wide_hbm.at[pl.ds(0, _CHUNK)], pad_v,
                                  sem).wait()

            def compact_row(i, _, k=k):
                for c in range(D // _LANES):
                    sl = pl.ds(c * _LANES, _LANES)
                    rows_v[k * _CHUNK + i, sl] = pad_v[i, sl]
                return 0

            lax.fori_loop(0, _CHUNK, compact_row, 0)

        pltpu.sync_copy(rows_v, out_hbm.at[pl.ds(base, b_per_w)])

    return sc_gather


def kernel(n_id, emb_table):
    B = n_id.shape[0]
    V, D = emb_table.shape
    wide = _build_widen(V, D)(emb_table)
    return _build_sc_gather(B, V, D)(n_id.astype(jnp.int32), wide)


# speedup vs baseline: 1.5960x; 1.5960x over previous
import functools

import jax
import jax.numpy as jnp
from jax import lax
from jax.experimental import pallas as pl
from jax.experimental.pallas import tpu as pltpu
from jax.experimental.pallas import tpu_sc as plsc

_LANES = 16
_SPLIT = 320


@functools.cache
def _build_sc_gather(B: int, V: int, D: int):
    info = plsc.get_sparse_core_info()
    nc, ns = info.num_cores, info.num_subcores
    nw = nc * ns
    assert B % (8 * nw) == 0, "batch must split 8-aligned across subcores"
    b_per_w = B // nw

    mesh = plsc.VectorSubcoreMesh(core_axis_name="c", subcore_axis_name="s")

    @functools.partial(
        pl.kernel,
        mesh=mesh,
        out_type=jax.ShapeDtypeStruct((B, D), jnp.float32),
        scratch_types=[
            pltpu.VMEM((b_per_w,), jnp.int32),
            pltpu.VMEM((_SPLIT, D), jnp.float32),
            pltpu.SemaphoreType.DMA,
            pltpu.SemaphoreType.DMA,
        ],
    )
    def sc_gather(n_id_hbm, tbl_hbm, out_hbm, idx_v, rows_v, sem_a, sem_b):
        wid = lax.axis_index("s") * nc + lax.axis_index("c")
        base = wid * b_per_w
        rest = b_per_w - _SPLIT
        pltpu.sync_copy(n_id_hbm.at[pl.ds(base, b_per_w)], idx_v)

        def fetch_a(jb, _):
            vec = idx_v[pl.ds(jb * _LANES, _LANES)]
            for lane in range(_LANES):
                row = vec[lane]
                pltpu.async_copy(
                    tbl_hbm.at[row], rows_v.at[jb * _LANES + lane], sem_a)
            return 0

        def fetch_b(jb, _):
            vec = idx_v[pl.ds(_SPLIT + jb * _LANES, _LANES)]
            for lane in range(_LANES):
                row = vec[lane]
                pltpu.async_copy(
                    tbl_hbm.at[row],
                    out_hbm.at[base + _SPLIT + jb * _LANES + lane], sem_b)
            return 0

        lax.fori_loop(0, _SPLIT // _LANES, fetch_a, 0)
        lax.fori_loop(0, rest // _LANES, fetch_b, 0)

        pltpu.make_async_copy(
            tbl_hbm.at[pl.ds(0, _SPLIT)], rows_v, sem_a).wait()
        pltpu.sync_copy(rows_v, out_hbm.at[pl.ds(base, _SPLIT)])
        pltpu.make_async_copy(
            tbl_hbm.at[pl.ds(0, rest)],
            out_hbm.at[pl.ds(base + _SPLIT, rest)], sem_b).wait()

    return sc_gather


def kernel(n_id, emb_table):
    B = n_id.shape[0]
    V, D = emb_table.shape
    sc_gather = _build_sc_gather(B, V, D)
    return sc_gather(n_id.astype(jnp.int32), emb_table)


# speedup vs baseline: 1.9867x; 1.2448x over previous
import functools

import jax
import jax.numpy as jnp
from jax import lax
from jax.experimental import pallas as pl
from jax.experimental.pallas import tpu as pltpu
from jax.experimental.pallas import tpu_sc as plsc

_LANES = 16


@functools.cache
def _build_sc_gather(B: int, V: int, D: int):
    info = plsc.get_sparse_core_info()
    nc, ns = info.num_cores, info.num_subcores
    nw = nc * ns
    assert B % (8 * nw) == 0, "batch must split 8-aligned across subcores"
    b_per_w = B // nw

    mesh = plsc.VectorSubcoreMesh(core_axis_name="c", subcore_axis_name="s")

    @functools.partial(
        pl.kernel,
        mesh=mesh,
        out_type=jax.ShapeDtypeStruct((B, D), jnp.float32),
        scratch_types=[
            pltpu.VMEM((b_per_w,), jnp.int32),
            pltpu.VMEM((b_per_w, D), jnp.float32),
            pltpu.SemaphoreType.DMA,
        ],
    )
    def sc_gather(n_id_hbm, tbl_hbm, out_hbm, idx_v, rows_v, sem):
        wid = lax.axis_index("s") * nc + lax.axis_index("c")
        base = wid * b_per_w
        pltpu.sync_copy(n_id_hbm.at[pl.ds(base, b_per_w)], idx_v)

        def fetch(jb, _):
            vec = idx_v[pl.ds(jb * _LANES, _LANES)]
            for lane in range(_LANES):
                row = vec[lane]
                pltpu.async_copy(
                    tbl_hbm.at[row], rows_v.at[jb * _LANES + lane], sem)
            return 0

        lax.fori_loop(0, b_per_w // _LANES, fetch, 0)

        pltpu.make_async_copy(
            tbl_hbm.at[pl.ds(0, b_per_w)], rows_v, sem).wait()
        pltpu.sync_copy(rows_v, out_hbm.at[pl.ds(base, b_per_w)])

    return sc_gather


def kernel(n_id, emb_table):
    B = n_id.shape[0]
    V, D = emb_table.shape
    sc_gather = _build_sc_gather(B, V, D)
    return sc_gather(n_id.astype(jnp.int32), emb_table)
